# baseline (speedup 1.0000x reference)
import functools

import jax
import jax.numpy as jnp
from jax import lax
from jax.experimental import pallas as pl
from jax.experimental.pallas import tpu as pltpu
from jax.experimental.pallas import tpu_sc as plsc

_LEVELS = (8, 8, 8, 16)
_BASIS = (1, 8, 64, 512)
_B = 262144
_D = 256

_NC = 2
_NS = 16
_NW = _NC * _NS
_BPW = _B // _NW
_CHUNK = 128
_NCHUNK = _BPW // _CHUNK
_NBUF = 2


def _index_body(zt_ref, p_ref, out_ref):
    shape = out_ref.shape
    big = jnp.zeros(shape, jnp.int32)
    for i in range(4):
        row = zt_ref[i:i + 1, :]
        col = jnp.zeros(shape, jnp.int32)
        for j in range(1, _LEVELS[i]):
            col = jnp.where(row >= p_ref[j, i], jnp.int32(j), col)
        big = big + col * jnp.int32(_BASIS[i])
    out_ref[...] = big


def _compute_indices(zt, percentiles, interpret=False):
    bt = 16384
    grid = _B // bt
    return pl.pallas_call(
        _index_body,
        grid=(grid,),
        in_specs=[
            pl.BlockSpec((4, bt), lambda b: (0, b)),
            pl.BlockSpec(memory_space=pltpu.SMEM),
        ],
        out_specs=pl.BlockSpec((1, bt), lambda b: (0, b)),
        out_shape=jax.ShapeDtypeStruct((1, _B), jnp.int32),
        interpret=interpret,
    )(zt, percentiles)


def _gather_body(cb_hbm, idx_hbm, out_hbm, idx_v, *rest):
    bufs = rest[:_NBUF]
    gsems = rest[_NBUF:2 * _NBUF]
    ssems = rest[2 * _NBUF:3 * _NBUF]
    wid = lax.axis_index("s") * _NC + lax.axis_index("c")
    base_row = wid * _BPW
    pltpu.sync_copy(idx_hbm.at[wid], idx_v)

    def group(gg, carry):
        g0 = gg * _NBUF
        gcopies = []
        for b in range(_NBUF):
            gcopies.append(
                pltpu.async_copy(cb_hbm.at[idx_v.at[g0 + b]], bufs[b], gsems[b]))
        scopies = []
        for b in range(_NBUF):
            gcopies[b].wait()
            row = base_row + (g0 + b) * _CHUNK
            scopies.append(
                pltpu.async_copy(bufs[b], out_hbm.at[pl.ds(row, _CHUNK)], ssems[b]))
        for b in range(_NBUF):
            scopies[b].wait()
        return carry

    lax.fori_loop(0, _NCHUNK // _NBUF, group, 0)


def _gather_call(codebook, idx3):
    mesh = plsc.VectorSubcoreMesh(core_axis_name="c", subcore_axis_name="s")
    scratch = [pltpu.VMEM((_NCHUNK, _CHUNK), jnp.int32)]
    scratch += [pltpu.VMEM((_CHUNK, _D), jnp.float32) for _ in range(_NBUF)]
    scratch += [pltpu.SemaphoreType.DMA for _ in range(2 * _NBUF)]
    run = pl.kernel(
        _gather_body,
        out_type=jax.ShapeDtypeStruct((_B, _D), jnp.float32),
        mesh=mesh,
        scratch_types=scratch,
    )
    return run(codebook, idx3)


def kernel(z, codebook, percentiles):
    zt = z.T
    idx2 = _compute_indices(zt, percentiles)
    idx3 = idx2.reshape(_NW, _NCHUNK, _CHUNK)
    quantized = _gather_call(codebook, idx3)
    return quantized, idx2.reshape(_B)
